# Initial kernel scaffold; baseline (speedup 1.0000x reference)
import jax
import jax.numpy as jnp
from jax.experimental import pallas as pl


def kernel(x, edge_index, Wc, W1, b1, W2, b2):
    raise NotImplementedError("write your pallas kernel here")


# speedup vs baseline: 30.3229x; 30.3229x over previous
import functools

import jax
import jax.numpy as jnp
from jax import lax
from jax.experimental import pallas as pl
from jax.experimental.pallas import tpu as pltpu
from jax.experimental.pallas import tpu_sc as plsc

N = 10000
E = 320000
D = 128

NP = 10240
EP = 327680
NW = 32
NCH = 80
CW = 128
RPT = NP // 16
HC = 40
BLK = 256
GRID = NP // BLK

_mesh = plsc.VectorSubcoreMesh(core_axis_name="c", subcore_axis_name="s")


@functools.partial(
    pl.kernel,
    mesh=_mesh,
    out_type=jax.ShapeDtypeStruct((2 * NP,), jnp.float32),
    scratch_types=[
        pltpu.VMEM((NCH, CW), jnp.int32),
        pltpu.VMEM((CW,), jnp.float32),
        pltpu.VMEM((RPT,), jnp.float32),
        pltpu.VMEM_SHARED((NP,), jnp.float32),
    ],
)
def _deg_kernel(dst_hbm, ones_hbm, zeros_hbm, cnt_hbm, idx_v, ones_v, buf_v,
                acc_sh):
    cid = lax.axis_index("c")
    sid = lax.axis_index("s")
    wid = cid * 16 + sid
    row0 = sid * RPT

    pltpu.sync_copy(dst_hbm.at[wid], idx_v)
    pltpu.sync_copy(ones_hbm, ones_v)
    pltpu.sync_copy(zeros_hbm, buf_v)
    pltpu.sync_copy(buf_v, acc_sh.at[pl.ds(row0, RPT)])
    plsc.subcore_barrier()

    @pl.loop(0, NCH)
    def _(j):
        pltpu.sync_copy(ones_v, acc_sh.at[idx_v.at[j]], add=True)

    plsc.subcore_barrier()
    pltpu.sync_copy(acc_sh.at[pl.ds(row0, RPT)], buf_v)
    pltpu.sync_copy(buf_v, cnt_hbm.at[pl.ds(cid * NP + row0, RPT)])


def _k2_body(c0_ref, c1_ref, x_ref, wct_ref, hn_ref):
    deg = 1.0 + c0_ref[...] + c1_ref[...]
    dis = lax.rsqrt(deg).reshape(BLK, 1)
    h = jnp.dot(x_ref[...], wct_ref[...], preferred_element_type=jnp.float32,
                precision=lax.Precision.HIGHEST)
    hn_ref[...] = h * dis


_k2 = pl.pallas_call(
    _k2_body,
    grid=(GRID,),
    in_specs=[
        pl.BlockSpec((BLK,), lambda i: (i,)),
        pl.BlockSpec((BLK,), lambda i: (i,)),
        pl.BlockSpec((BLK, D), lambda i: (i, 0)),
        pl.BlockSpec((D, D), lambda i: (0, 0)),
    ],
    out_specs=pl.BlockSpec((BLK, D), lambda i: (i, 0)),
    out_shape=jax.ShapeDtypeStruct((NP, D), jnp.float32),
)


@functools.partial(
    pl.kernel,
    mesh=_mesh,
    out_type=jax.ShapeDtypeStruct((2, NP, D), jnp.float32),
    scratch_types=[
        pltpu.VMEM((HC, CW), jnp.int32),
        pltpu.VMEM((HC, CW), jnp.int32),
        pltpu.VMEM((CW, D), jnp.float32),
        pltpu.VMEM((CW, D), jnp.float32),
        pltpu.VMEM_SHARED((NP, D), jnp.float32),
        pltpu.SemaphoreType.DMA,
        pltpu.SemaphoreType.DMA,
    ],
)
def _scatter_kernel(src_hbm, dst_hbm, hn_hbm, z2_hbm, part_hbm, sidx, didx,
                    rows0, rows1, acc_sh, sem0, sem1):
    cid = lax.axis_index("c")
    sid = lax.axis_index("s")
    wid = cid * 16 + sid
    row0 = sid * RPT

    pltpu.sync_copy(z2_hbm, rows0)

    @pl.loop(0, RPT // CW)
    def _(k):
        pltpu.sync_copy(rows0, acc_sh.at[pl.ds(row0 + k * CW, CW)])

    plsc.subcore_barrier()

    @pl.loop(0, NCH // HC)
    def _(h):
        pltpu.sync_copy(src_hbm.at[wid, pl.ds(h * HC, HC)], sidx)
        pltpu.sync_copy(dst_hbm.at[wid, pl.ds(h * HC, HC)], didx)

        pltpu.async_copy(hn_hbm.at[sidx.at[0]], rows0, sem0).wait()

        @pl.loop(0, HC - 2, step=2)
        def _(j):
            cp1 = pltpu.async_copy(hn_hbm.at[sidx.at[j + 1]], rows1, sem1)
            pltpu.sync_copy(rows0, acc_sh.at[didx.at[j]], add=True)
            cp1.wait()
            cp0 = pltpu.async_copy(hn_hbm.at[sidx.at[j + 2]], rows0, sem0)
            pltpu.sync_copy(rows1, acc_sh.at[didx.at[j + 1]], add=True)
            cp0.wait()

        pltpu.async_copy(hn_hbm.at[sidx.at[HC - 1]], rows1, sem1).wait()
        pltpu.sync_copy(rows0, acc_sh.at[didx.at[HC - 2]], add=True)
        pltpu.sync_copy(rows1, acc_sh.at[didx.at[HC - 1]], add=True)

    plsc.subcore_barrier()

    @pl.loop(0, RPT // CW)
    def _(k):
        sl = pl.ds(row0 + k * CW, CW)
        pltpu.sync_copy(acc_sh.at[sl], rows0)
        pltpu.sync_copy(rows0, part_hbm.at[cid].at[sl])


def _k4_body(p_ref, hn_ref, c0_ref, c1_ref, w1t_ref, b1_ref, w2t_ref, b2_ref,
             y_ref, g_ref):
    p = p_ref[...]
    s = p[0] + p[1] + hn_ref[...]
    deg = 1.0 + c0_ref[...] + c1_ref[...]
    dis = lax.rsqrt(deg).reshape(BLK, 1)
    g = jnp.maximum(dis * s, 0.0)
    h1 = jnp.maximum(
        jnp.dot(g, w1t_ref[...], preferred_element_type=jnp.float32,
                precision=lax.Precision.HIGHEST) + b1_ref[...], 0.0)
    y_ref[...] = jnp.dot(h1, w2t_ref[...], preferred_element_type=jnp.float32,
                         precision=lax.Precision.HIGHEST) + b2_ref[...]
    g_ref[...] = g


_k4 = pl.pallas_call(
    _k4_body,
    grid=(GRID,),
    in_specs=[
        pl.BlockSpec((2, BLK, D), lambda i: (0, i, 0)),
        pl.BlockSpec((BLK, D), lambda i: (i, 0)),
        pl.BlockSpec((BLK,), lambda i: (i,)),
        pl.BlockSpec((BLK,), lambda i: (i,)),
        pl.BlockSpec((D, D), lambda i: (0, 0)),
        pl.BlockSpec((1, D), lambda i: (0, 0)),
        pl.BlockSpec((D, D), lambda i: (0, 0)),
        pl.BlockSpec((1, D), lambda i: (0, 0)),
    ],
    out_specs=[
        pl.BlockSpec((BLK, D), lambda i: (i, 0)),
        pl.BlockSpec((BLK, D), lambda i: (i, 0)),
    ],
    out_shape=[
        jax.ShapeDtypeStruct((NP, D), jnp.float32),
        jax.ShapeDtypeStruct((NP, D), jnp.float32),
    ],
)


def kernel(x, edge_index, Wc, W1, b1, W2, b2):
    pad = EP - E
    padidx = (jnp.arange(pad, dtype=jnp.int32) % 240) + N
    srcp = jnp.concatenate([edge_index[0], padidx]).reshape(NW, NCH, CW)
    dstp = jnp.concatenate([edge_index[1], padidx]).reshape(NW, NCH, CW)

    ones1 = jnp.ones((CW,), jnp.float32)
    zeros1 = jnp.zeros((RPT,), jnp.float32)
    zeros2 = jnp.zeros((CW, D), jnp.float32)

    counts = _deg_kernel(dstp, ones1, zeros1)
    c0 = counts[:NP]
    c1 = counts[NP:]

    x_pad = jnp.pad(x, ((0, NP - N), (0, 0)))
    hn = _k2(c0, c1, x_pad, Wc.T)

    partials = _scatter_kernel(srcp, dstp, hn, zeros2)

    y_full, g_full = _k4(partials, hn, c0, c1, W1.T, b1[None, :], W2.T,
                         b2[None, :])
    y = y_full[:N].reshape(N, 1, D)
    return (y, g_full[:N])


# speedup vs baseline: 33.8921x; 1.1177x over previous
import functools

import jax
import jax.numpy as jnp
import numpy as np
from jax import lax
from jax.experimental import pallas as pl
from jax.experimental.pallas import tpu as pltpu
from jax.experimental.pallas import tpu_sc as plsc

N = 10000
E = 320000
D = 128

NP = 10240
EP = 327680
NW = 32
NCH = 80
CW = 128
RPT = NP // 16
HC = 40
BLK = 1024
GRID = NP // BLK

_PAD_IDX = np.asarray((np.arange(EP - E) % 240) + N, dtype=np.int32)

_mesh = plsc.VectorSubcoreMesh(core_axis_name="c", subcore_axis_name="s")


@functools.partial(
    pl.kernel,
    mesh=_mesh,
    out_type=jax.ShapeDtypeStruct((2 * NP,), jnp.float32),
    scratch_types=[
        pltpu.VMEM((NCH, CW), jnp.int32),
        pltpu.VMEM((CW,), jnp.float32),
        pltpu.VMEM((RPT,), jnp.float32),
        pltpu.VMEM_SHARED((NP,), jnp.float32),
    ],
)
def _deg_kernel(dst_hbm, ones_hbm, zeros_hbm, cnt_hbm, idx_v, ones_v, buf_v,
                acc_sh):
    cid = lax.axis_index("c")
    sid = lax.axis_index("s")
    wid = cid * 16 + sid
    row0 = sid * RPT

    pltpu.sync_copy(dst_hbm.at[wid], idx_v)
    pltpu.sync_copy(ones_hbm, ones_v)
    pltpu.sync_copy(zeros_hbm, buf_v)
    pltpu.sync_copy(buf_v, acc_sh.at[pl.ds(row0, RPT)])
    plsc.subcore_barrier()

    @pl.loop(0, NCH)
    def _(j):
        pltpu.sync_copy(ones_v, acc_sh.at[idx_v.at[j]], add=True)

    plsc.subcore_barrier()
    pltpu.sync_copy(acc_sh.at[pl.ds(row0, RPT)], buf_v)
    pltpu.sync_copy(buf_v, cnt_hbm.at[pl.ds(cid * NP + row0, RPT)])


def _k2_body(c0_ref, c1_ref, x_ref, wct_ref, hn_ref):
    deg = 1.0 + c0_ref[...] + c1_ref[...]
    dis = lax.rsqrt(deg).reshape(BLK, 1)
    h = jnp.dot(x_ref[...], wct_ref[...], preferred_element_type=jnp.float32,
                precision=lax.Precision.HIGHEST)
    hn_ref[...] = h * dis


_k2 = pl.pallas_call(
    _k2_body,
    grid=(GRID,),
    in_specs=[
        pl.BlockSpec((BLK,), lambda i: (i,)),
        pl.BlockSpec((BLK,), lambda i: (i,)),
        pl.BlockSpec((BLK, D), lambda i: (i, 0)),
        pl.BlockSpec((D, D), lambda i: (0, 0)),
    ],
    out_specs=pl.BlockSpec((BLK, D), lambda i: (i, 0)),
    out_shape=jax.ShapeDtypeStruct((NP, D), jnp.float32),
)


@functools.partial(
    pl.kernel,
    mesh=_mesh,
    out_type=jax.ShapeDtypeStruct((2, NP, D), jnp.float32),
    scratch_types=[
        pltpu.VMEM((HC, CW), jnp.int32),
        pltpu.VMEM((HC, CW), jnp.int32),
        pltpu.VMEM((CW, D), jnp.float32),
        pltpu.VMEM((CW, D), jnp.float32),
        pltpu.VMEM_SHARED((NP, D), jnp.float32),
        pltpu.SemaphoreType.DMA,
        pltpu.SemaphoreType.DMA,
        pltpu.SemaphoreType.DMA,
        pltpu.SemaphoreType.DMA,
    ],
)
def _scatter_kernel(src_hbm, dst_hbm, hn_hbm, z2_hbm, part_hbm, sidx, didx,
                    rows0, rows1, acc_sh, gsem0, gsem1, ssem0, ssem1):
    cid = lax.axis_index("c")
    sid = lax.axis_index("s")
    wid = cid * 16 + sid
    row0 = sid * RPT

    pltpu.sync_copy(z2_hbm, rows0)

    @pl.loop(0, RPT // CW)
    def _(k):
        pltpu.sync_copy(rows0, acc_sh.at[pl.ds(row0 + k * CW, CW)])

    plsc.subcore_barrier()

    def _gather(j, buf, sem):
        pltpu.async_copy(hn_hbm.at[sidx.at[j]], buf, sem)

    def _scat(j, buf, sem):
        pltpu.async_copy(buf, acc_sh.at[didx.at[j]], sem, add=True)

    def _drain_g(buf, sem):
        pltpu.make_async_copy(hn_hbm.at[sidx.at[0]], buf, sem).wait()

    def _drain_s(buf, sem):
        pltpu.make_async_copy(buf, acc_sh.at[didx.at[0]], sem).wait()

    @pl.loop(0, NCH // HC)
    def _(h):
        pltpu.sync_copy(src_hbm.at[wid, pl.ds(h * HC, HC)], sidx)
        pltpu.sync_copy(dst_hbm.at[wid, pl.ds(h * HC, HC)], didx)

        _gather(0, rows0, gsem0)
        _gather(1, rows1, gsem1)

        @pl.loop(0, HC - 2, step=2)
        def _(j):
            _drain_g(rows0, gsem0)
            _scat(j, rows0, ssem0)
            _drain_g(rows1, gsem1)
            _scat(j + 1, rows1, ssem1)
            _drain_s(rows0, ssem0)
            _gather(j + 2, rows0, gsem0)
            _drain_s(rows1, ssem1)
            _gather(j + 3, rows1, gsem1)

        _drain_g(rows0, gsem0)
        _scat(HC - 2, rows0, ssem0)
        _drain_g(rows1, gsem1)
        _scat(HC - 1, rows1, ssem1)
        _drain_s(rows0, ssem0)
        _drain_s(rows1, ssem1)

    plsc.subcore_barrier()

    @pl.loop(0, RPT // CW)
    def _(k):
        sl = pl.ds(row0 + k * CW, CW)
        pltpu.sync_copy(acc_sh.at[sl], rows0)
        pltpu.sync_copy(rows0, part_hbm.at[cid].at[sl])


def _k4_body(p_ref, hn_ref, c0_ref, c1_ref, w1t_ref, b1_ref, w2t_ref, b2_ref,
             y_ref, g_ref):
    p = p_ref[...]
    s = p[0] + p[1] + hn_ref[...]
    deg = 1.0 + c0_ref[...] + c1_ref[...]
    dis = lax.rsqrt(deg).reshape(BLK, 1)
    g = jnp.maximum(dis * s, 0.0)
    h1 = jnp.maximum(
        jnp.dot(g, w1t_ref[...], preferred_element_type=jnp.float32,
                precision=lax.Precision.HIGHEST) + b1_ref[...], 0.0)
    y_ref[...] = jnp.dot(h1, w2t_ref[...], preferred_element_type=jnp.float32,
                         precision=lax.Precision.HIGHEST) + b2_ref[...]
    g_ref[...] = g


_k4 = pl.pallas_call(
    _k4_body,
    grid=(GRID,),
    in_specs=[
        pl.BlockSpec((2, BLK, D), lambda i: (0, i, 0)),
        pl.BlockSpec((BLK, D), lambda i: (i, 0)),
        pl.BlockSpec((BLK,), lambda i: (i,)),
        pl.BlockSpec((BLK,), lambda i: (i,)),
        pl.BlockSpec((D, D), lambda i: (0, 0)),
        pl.BlockSpec((1, D), lambda i: (0, 0)),
        pl.BlockSpec((D, D), lambda i: (0, 0)),
        pl.BlockSpec((1, D), lambda i: (0, 0)),
    ],
    out_specs=[
        pl.BlockSpec((BLK, D), lambda i: (i, 0)),
        pl.BlockSpec((BLK, D), lambda i: (i, 0)),
    ],
    out_shape=[
        jax.ShapeDtypeStruct((N, D), jnp.float32),
        jax.ShapeDtypeStruct((N, D), jnp.float32),
    ],
)


def kernel(x, edge_index, Wc, W1, b1, W2, b2):
    padidx = jnp.asarray(_PAD_IDX)
    srcp = jnp.concatenate([edge_index[0], padidx]).reshape(NW, NCH, CW)
    dstp = jnp.concatenate([edge_index[1], padidx]).reshape(NW, NCH, CW)

    ones1 = jnp.ones((CW,), jnp.float32)
    zeros1 = jnp.zeros((RPT,), jnp.float32)
    zeros2 = jnp.zeros((CW, D), jnp.float32)

    counts = _deg_kernel(dstp, ones1, zeros1)
    c0 = counts[:NP]
    c1 = counts[NP:]

    hn = _k2(c0, c1, x, Wc.T)

    partials = _scatter_kernel(srcp, dstp, hn, zeros2)

    y2d, g = _k4(partials, hn, c0, c1, W1.T, b1[None, :], W2.T, b2[None, :])
    return (y2d.reshape(N, 1, D), g)


# speedup vs baseline: 38.8407x; 1.1460x over previous
import functools

import jax
import jax.numpy as jnp
import numpy as np
from jax import lax
from jax.experimental import pallas as pl
from jax.experimental.pallas import tpu as pltpu
from jax.experimental.pallas import tpu_sc as plsc

N = 10000
E = 320000
D = 128

NP = 10240
EP = 327680
NW = 32
NCH = 80
CW = 128
RPT = NP // 16
HC = 40
BLK = 1024
GRID = NP // BLK

_PAD_IDX = np.asarray((np.arange(EP - E) % 240) + N, dtype=np.int32)

_mesh = plsc.VectorSubcoreMesh(core_axis_name="c", subcore_axis_name="s")


@functools.partial(
    pl.kernel,
    mesh=_mesh,
    out_type=jax.ShapeDtypeStruct((2 * NP,), jnp.float32),
    scratch_types=[
        pltpu.VMEM((NCH, CW), jnp.int32),
        pltpu.VMEM((CW,), jnp.float32),
        pltpu.VMEM((RPT,), jnp.float32),
        pltpu.VMEM_SHARED((NP,), jnp.float32),
        pltpu.SemaphoreType.DMA,
    ],
)
def _deg_kernel(dst_hbm, ones_hbm, zeros_hbm, cnt_hbm, idx_v, ones_v, buf_v,
                acc_sh, sem):
    cid = lax.axis_index("c")
    sid = lax.axis_index("s")
    wid = cid * 16 + sid
    row0 = sid * RPT

    pltpu.sync_copy(dst_hbm.at[wid], idx_v)
    pltpu.sync_copy(ones_hbm, ones_v)
    pltpu.sync_copy(zeros_hbm, buf_v)
    pltpu.sync_copy(buf_v, acc_sh.at[pl.ds(row0, RPT)])
    plsc.subcore_barrier()

    @pl.loop(0, NCH)
    def _(j):
        pltpu.async_copy(ones_v, acc_sh.at[idx_v.at[j]], sem, add=True)

    @pl.loop(0, NCH)
    def _(j):
        pltpu.make_async_copy(ones_v, acc_sh.at[idx_v.at[0]], sem).wait()

    plsc.subcore_barrier()
    pltpu.sync_copy(acc_sh.at[pl.ds(row0, RPT)], buf_v)
    pltpu.sync_copy(buf_v, cnt_hbm.at[pl.ds(cid * NP + row0, RPT)])


def _k2_body(c0_ref, c1_ref, x_ref, wct_ref, hn_ref):
    deg = 1.0 + c0_ref[...] + c1_ref[...]
    dis = lax.rsqrt(deg).reshape(BLK, 1)
    h = jnp.dot(x_ref[...], wct_ref[...], preferred_element_type=jnp.float32)
    hn_ref[...] = h * dis


_k2 = pl.pallas_call(
    _k2_body,
    grid=(GRID,),
    in_specs=[
        pl.BlockSpec((BLK,), lambda i: (i,)),
        pl.BlockSpec((BLK,), lambda i: (i,)),
        pl.BlockSpec((BLK, D), lambda i: (i, 0)),
        pl.BlockSpec((D, D), lambda i: (0, 0)),
    ],
    out_specs=pl.BlockSpec((BLK, D), lambda i: (i, 0)),
    out_shape=jax.ShapeDtypeStruct((NP, D), jnp.float32),
)


@functools.partial(
    pl.kernel,
    mesh=_mesh,
    out_type=[
        jax.ShapeDtypeStruct((NP, D), jnp.float32),
        jax.ShapeDtypeStruct((NP, D), jnp.float32),
    ],
    scratch_types=[
        pltpu.VMEM((HC, CW), jnp.int32),
        pltpu.VMEM((HC, CW), jnp.int32),
        pltpu.VMEM((CW, D), jnp.float32),
        pltpu.VMEM((CW, D), jnp.float32),
        pltpu.VMEM_SHARED((NP, D), jnp.float32),
        pltpu.SemaphoreType.DMA,
        pltpu.SemaphoreType.DMA,
    ],
)
def _scatter_kernel(src_hbm, dst_hbm, hn_hbm, z2_hbm, p0_hbm, p1_hbm, sidx,
                    didx, rows0, rows1, acc_sh, sem0, sem1):
    cid = lax.axis_index("c")
    sid = lax.axis_index("s")
    wid = cid * 16 + sid
    row0 = sid * RPT

    pltpu.sync_copy(z2_hbm, rows0)

    @pl.loop(0, RPT // CW)
    def _(k):
        pltpu.sync_copy(rows0, acc_sh.at[pl.ds(row0 + k * CW, CW)])

    plsc.subcore_barrier()

    @pl.loop(0, NCH // HC)
    def _(h):
        pltpu.sync_copy(src_hbm.at[wid, pl.ds(h * HC, HC)], sidx)
        pltpu.sync_copy(dst_hbm.at[wid, pl.ds(h * HC, HC)], didx)

        pltpu.async_copy(hn_hbm.at[sidx.at[0]], rows0, sem0).wait()

        @pl.loop(0, HC - 2, step=2)
        def _(j):
            cp1 = pltpu.async_copy(hn_hbm.at[sidx.at[j + 1]], rows1, sem1)
            pltpu.sync_copy(rows0, acc_sh.at[didx.at[j]], add=True)
            cp1.wait()
            cp0 = pltpu.async_copy(hn_hbm.at[sidx.at[j + 2]], rows0, sem0)
            pltpu.sync_copy(rows1, acc_sh.at[didx.at[j + 1]], add=True)
            cp0.wait()

        pltpu.async_copy(hn_hbm.at[sidx.at[HC - 1]], rows1, sem1).wait()
        pltpu.sync_copy(rows0, acc_sh.at[didx.at[HC - 2]], add=True)
        pltpu.sync_copy(rows1, acc_sh.at[didx.at[HC - 1]], add=True)

    plsc.subcore_barrier()

    @pl.loop(0, RPT // CW)
    def _(k):
        sl = pl.ds(row0 + k * CW, CW)
        pltpu.sync_copy(acc_sh.at[sl], rows0)

        @pl.when(cid == 0)
        def _():
            pltpu.sync_copy(rows0, p0_hbm.at[sl])

        @pl.when(cid == 1)
        def _():
            pltpu.sync_copy(rows0, p1_hbm.at[sl])


def _k4_body(p0_ref, p1_ref, hn_ref, c0_ref, c1_ref, w1t_ref, b1_ref,
             w2t_ref, b2_ref, y_ref, g_ref):
    s = p0_ref[...] + p1_ref[...] + hn_ref[...]
    deg = 1.0 + c0_ref[...] + c1_ref[...]
    dis = lax.rsqrt(deg).reshape(BLK, 1)
    g = jnp.maximum(dis * s, 0.0)
    h1 = jnp.maximum(
        jnp.dot(g, w1t_ref[...], preferred_element_type=jnp.float32)
        + b1_ref[...], 0.0)
    y_ref[...] = (jnp.dot(h1, w2t_ref[...], preferred_element_type=jnp.float32)
                  + b2_ref[...])
    g_ref[...] = g


_k4 = pl.pallas_call(
    _k4_body,
    grid=(GRID,),
    in_specs=[
        pl.BlockSpec((BLK, D), lambda i: (i, 0)),
        pl.BlockSpec((BLK, D), lambda i: (i, 0)),
        pl.BlockSpec((BLK, D), lambda i: (i, 0)),
        pl.BlockSpec((BLK,), lambda i: (i,)),
        pl.BlockSpec((BLK,), lambda i: (i,)),
        pl.BlockSpec((D, D), lambda i: (0, 0)),
        pl.BlockSpec((1, D), lambda i: (0, 0)),
        pl.BlockSpec((D, D), lambda i: (0, 0)),
        pl.BlockSpec((1, D), lambda i: (0, 0)),
    ],
    out_specs=[
        pl.BlockSpec((BLK, D), lambda i: (i, 0)),
        pl.BlockSpec((BLK, D), lambda i: (i, 0)),
    ],
    out_shape=[
        jax.ShapeDtypeStruct((N, D), jnp.float32),
        jax.ShapeDtypeStruct((N, D), jnp.float32),
    ],
)


def kernel(x, edge_index, Wc, W1, b1, W2, b2):
    padidx = jnp.asarray(_PAD_IDX)
    srcp = jnp.concatenate([edge_index[0], padidx]).reshape(NW, NCH, CW)
    dstp = jnp.concatenate([edge_index[1], padidx]).reshape(NW, NCH, CW)

    ones1 = jnp.ones((CW,), jnp.float32)
    zeros1 = jnp.zeros((RPT,), jnp.float32)
    zeros2 = jnp.zeros((CW, D), jnp.float32)

    counts = _deg_kernel(dstp, ones1, zeros1)
    c0 = counts[:NP]
    c1 = counts[NP:]

    hn = _k2(c0, c1, x, Wc.T)

    p0, p1 = _scatter_kernel(srcp, dstp, hn, zeros2)

    y2d, g = _k4(p0, p1, hn, c0, c1, W1.T, b1[None, :], W2.T, b2[None, :])
    return (y2d.reshape(N, 1, D), g)


# speedup vs baseline: 38.9906x; 1.0039x over previous
import functools

import jax
import jax.numpy as jnp
import numpy as np
from jax import lax
from jax.experimental import pallas as pl
from jax.experimental.pallas import tpu as pltpu
from jax.experimental.pallas import tpu_sc as plsc

N = 10000
E = 320000
D = 128

NP = 10240
EP = 327680
NW = 32
EPW = EP // NW
NCH = 80
CW = 128
RPT = NP // 16
HC = 40
BLK = 1024
GRID = NP // BLK

_PAD_IDX = np.asarray((np.arange(EP - E) % 240) + N, dtype=np.int32)

_mesh = plsc.VectorSubcoreMesh(core_axis_name="c", subcore_axis_name="s")


@functools.partial(
    pl.kernel,
    mesh=_mesh,
    out_type=jax.ShapeDtypeStruct((2 * NP,), jnp.float32),
    scratch_types=[
        pltpu.VMEM((EPW,), jnp.int32),
        pltpu.VMEM((NCH, CW), jnp.int32),
        pltpu.VMEM((CW,), jnp.float32),
        pltpu.VMEM((RPT,), jnp.float32),
        pltpu.VMEM_SHARED((NP,), jnp.float32),
        pltpu.SemaphoreType.DMA,
    ],
)
def _deg_kernel(dst_hbm, ones_hbm, zeros_hbm, cnt_hbm, idx1_v, idx_v, ones_v,
                buf_v, acc_sh, sem):
    cid = lax.axis_index("c")
    sid = lax.axis_index("s")
    wid = cid * 16 + sid
    row0 = sid * RPT

    pltpu.sync_copy(dst_hbm.at[pl.ds(wid * EPW, EPW)], idx1_v)
    pltpu.sync_copy(ones_hbm, ones_v)
    pltpu.sync_copy(zeros_hbm, buf_v)
    pltpu.sync_copy(buf_v, acc_sh.at[pl.ds(row0, RPT)])

    @pl.loop(0, NCH)
    def _(j):
        @pl.loop(0, CW // 16)
        def _(k):
            idx_v[j, pl.ds(k * 16, 16)] = idx1_v[pl.ds(j * CW + k * 16, 16)]

    plsc.subcore_barrier()

    @pl.loop(0, NCH)
    def _(j):
        pltpu.async_copy(ones_v, acc_sh.at[idx_v.at[j]], sem, add=True)

    @pl.loop(0, NCH)
    def _(j):
        pltpu.make_async_copy(ones_v, acc_sh.at[idx_v.at[0]], sem).wait()

    plsc.subcore_barrier()
    pltpu.sync_copy(acc_sh.at[pl.ds(row0, RPT)], buf_v)
    pltpu.sync_copy(buf_v, cnt_hbm.at[pl.ds(cid * NP + row0, RPT)])


def _k2_body(c0_ref, c1_ref, x_ref, wct_ref, hn_ref):
    deg = 1.0 + c0_ref[...] + c1_ref[...]
    dis = lax.rsqrt(deg).reshape(BLK, 1)
    h = jnp.dot(x_ref[...], wct_ref[...], preferred_element_type=jnp.float32)
    hn_ref[...] = h * dis


_k2 = pl.pallas_call(
    _k2_body,
    grid=(GRID,),
    in_specs=[
        pl.BlockSpec((BLK,), lambda i: (i,)),
        pl.BlockSpec((BLK,), lambda i: (i,)),
        pl.BlockSpec((BLK, D), lambda i: (i, 0)),
        pl.BlockSpec((D, D), lambda i: (0, 0)),
    ],
    out_specs=pl.BlockSpec((BLK, D), lambda i: (i, 0)),
    out_shape=jax.ShapeDtypeStruct((NP, D), jnp.float32),
)


@functools.partial(
    pl.kernel,
    mesh=_mesh,
    out_type=[
        jax.ShapeDtypeStruct((NP, D), jnp.float32),
        jax.ShapeDtypeStruct((NP, D), jnp.float32),
    ],
    scratch_types=[
        pltpu.VMEM((HC * CW,), jnp.int32),
        pltpu.VMEM((HC * CW,), jnp.int32),
        pltpu.VMEM((2, CW), jnp.int32),
        pltpu.VMEM((CW, D), jnp.float32),
        pltpu.VMEM((CW, D), jnp.float32),
        pltpu.VMEM_SHARED((NP, D), jnp.float32),
        pltpu.SemaphoreType.DMA,
        pltpu.SemaphoreType.DMA,
    ],
)
def _scatter_kernel(src_hbm, dst_hbm, hn_hbm, z2_hbm, p0_hbm, p1_hbm, sidx,
                    didx, dstage, rows0, rows1, acc_sh, sem0, sem1):
    cid = lax.axis_index("c")
    sid = lax.axis_index("s")
    wid = cid * 16 + sid
    row0 = sid * RPT

    pltpu.sync_copy(z2_hbm, rows0)

    @pl.loop(0, RPT // CW)
    def _(k):
        pltpu.sync_copy(rows0, acc_sh.at[pl.ds(row0 + k * CW, CW)])

    plsc.subcore_barrier()

    def _stage(j, slot):
        @pl.loop(0, CW // 16)
        def _(k):
            dstage[slot, pl.ds(k * 16, 16)] = didx[pl.ds(j * CW + k * 16, 16)]

    def _gidx(j):
        return sidx.at[pl.ds(j * CW, CW)]

    @pl.loop(0, NCH // HC)
    def _(h):
        pltpu.sync_copy(src_hbm.at[pl.ds(wid * EPW + h * HC * CW, HC * CW)],
                        sidx)
        pltpu.sync_copy(dst_hbm.at[pl.ds(wid * EPW + h * HC * CW, HC * CW)],
                        didx)

        _stage(0, 0)
        pltpu.async_copy(hn_hbm.at[_gidx(0)], rows0, sem0).wait()

        @pl.loop(0, HC - 2, step=2)
        def _(j):
            cp1 = pltpu.async_copy(hn_hbm.at[_gidx(j + 1)], rows1, sem1)
            _stage(j + 1, 1)
            pltpu.sync_copy(rows0, acc_sh.at[dstage.at[0]], add=True)
            cp1.wait()
            cp0 = pltpu.async_copy(hn_hbm.at[_gidx(j + 2)], rows0, sem0)
            _stage(j + 2, 0)
            pltpu.sync_copy(rows1, acc_sh.at[dstage.at[1]], add=True)
            cp0.wait()

        cpl = pltpu.async_copy(hn_hbm.at[_gidx(HC - 1)], rows1, sem1)
        _stage(HC - 1, 1)
        pltpu.sync_copy(rows0, acc_sh.at[dstage.at[0]], add=True)
        cpl.wait()
        pltpu.sync_copy(rows1, acc_sh.at[dstage.at[1]], add=True)

    plsc.subcore_barrier()

    @pl.loop(0, RPT // CW)
    def _(k):
        sl = pl.ds(row0 + k * CW, CW)
        pltpu.sync_copy(acc_sh.at[sl], rows0)

        @pl.when(cid == 0)
        def _():
            pltpu.sync_copy(rows0, p0_hbm.at[sl])

        @pl.when(cid == 1)
        def _():
            pltpu.sync_copy(rows0, p1_hbm.at[sl])


def _k4_body(p0_ref, p1_ref, hn_ref, c0_ref, c1_ref, w1t_ref, b1_ref,
             w2t_ref, b2_ref, y_ref, g_ref):
    s = p0_ref[...] + p1_ref[...] + hn_ref[...]
    deg = 1.0 + c0_ref[...] + c1_ref[...]
    dis = lax.rsqrt(deg).reshape(BLK, 1)
    g = jnp.maximum(dis * s, 0.0)
    h1 = jnp.maximum(
        jnp.dot(g, w1t_ref[...], preferred_element_type=jnp.float32)
        + b1_ref[...], 0.0)
    y = (jnp.dot(h1, w2t_ref[...], preferred_element_type=jnp.float32)
         + b2_ref[...])
    y_ref[...] = y.reshape(BLK, 1, D)
    g_ref[...] = g


_k4 = pl.pallas_call(
    _k4_body,
    grid=(GRID,),
    in_specs=[
        pl.BlockSpec((BLK, D), lambda i: (i, 0)),
        pl.BlockSpec((BLK, D), lambda i: (i, 0)),
        pl.BlockSpec((BLK, D), lambda i: (i, 0)),
        pl.BlockSpec((BLK,), lambda i: (i,)),
        pl.BlockSpec((BLK,), lambda i: (i,)),
        pl.BlockSpec((D, D), lambda i: (0, 0)),
        pl.BlockSpec((1, D), lambda i: (0, 0)),
        pl.BlockSpec((D, D), lambda i: (0, 0)),
        pl.BlockSpec((1, D), lambda i: (0, 0)),
    ],
    out_specs=[
        pl.BlockSpec((BLK, 1, D), lambda i: (i, 0, 0)),
        pl.BlockSpec((BLK, D), lambda i: (i, 0)),
    ],
    out_shape=[
        jax.ShapeDtypeStruct((N, 1, D), jnp.float32),
        jax.ShapeDtypeStruct((N, D), jnp.float32),
    ],
)


def kernel(x, edge_index, Wc, W1, b1, W2, b2):
    padidx = jnp.asarray(_PAD_IDX)
    srcp = jnp.concatenate([edge_index[0], padidx])
    dstp = jnp.concatenate([edge_index[1], padidx])

    ones1 = jnp.ones((CW,), jnp.float32)
    zeros1 = jnp.zeros((RPT,), jnp.float32)
    zeros2 = jnp.zeros((CW, D), jnp.float32)

    counts = _deg_kernel(dstp, ones1, zeros1)
    c0 = counts[:NP]
    c1 = counts[NP:]

    hn = _k2(c0, c1, x, Wc.T)

    p0, p1 = _scatter_kernel(srcp, dstp, hn, zeros2)

    y, g = _k4(p0, p1, hn, c0, c1, W1.T, b1[None, :], W2.T, b2[None, :])
    return (y, g)
